# baseline (device time: 191017 ns/iter reference)
import os

import jax
import jax.numpy as jnp
from jax import lax
from jax.experimental import pallas as pl
from jax.experimental.pallas import tpu as pltpu

N_DEV = 16
H = 8
SUBS = int(os.environ.get("KSUBS", "4"))

_KMODE = int(os.environ.get("KMODE", "0"))


def kernel(x, w_mat):
    m_per, k = x.shape
    _, n_per = w_mat.shape

    def body(x_ref, w_ref, out_ref, xbf, wbf, rbuf, lbuf, abuf,
             r_send, r_recv, l_send, l_recv, a_send, a_recv):
        me = lax.axis_index("i")
        left = (me + N_DEV - 1) % N_DEV
        right = (me + 1) % N_DEV

        barrier = pltpu.get_barrier_semaphore()
        for nbr in (left, right):
            pl.semaphore_signal(barrier, inc=1, device_id=(nbr,),
                                device_id_type=pl.DeviceIdType.MESH)
        pl.semaphore_wait(barrier, 2)

        xbf[...] = x_ref[...].astype(jnp.bfloat16)
        wbf[...] = w_ref[...].astype(jnp.bfloat16)

        def start_send(src, dst, ssem, rsem, dev):
            rdma = pltpu.make_async_remote_copy(
                src_ref=src, dst_ref=dst, send_sem=ssem, recv_sem=rsem,
                device_id=(dev,), device_id_type=pl.DeviceIdType.MESH)
            rdma.start()
            return rdma

        def wait_recv(dst, rsem):
            pltpu.make_async_remote_copy(
                src_ref=dst, dst_ref=dst, send_sem=rsem, recv_sem=rsem,
                device_id=(me,), device_id_type=pl.DeviceIdType.MESH,
            ).wait_recv()

        half = m_per // 2

        def gemm_store(chunk, row0, rows):
            if _KMODE >= 2:
                return
            y = jnp.dot(chunk, wbf[...], preferred_element_type=jnp.float32)
            out_ref[pl.ds(row0, rows), :] = jnp.maximum(y, 0.0)

        def sub_rows(h, s, is_left):
            if h < H - 1:
                sub = m_per // SUBS
                return ((SUBS - 1 - s) if is_left else s) * sub, sub
            q = half // SUBS
            if is_left:
                return m_per - (s + 1) * q, q
            return s * q, q

        sends = []
        for s in range(SUBS):
            r0, nr = sub_rows(0, s, False)
            sends.append(start_send(xbf.at[pl.ds(r0, nr)],
                                    rbuf.at[0, pl.ds(r0, nr)],
                                    r_send.at[0, s], r_recv.at[0, s], right))
            r0, nr = sub_rows(0, s, True)
            sends.append(start_send(xbf.at[pl.ds(r0, nr)],
                                    lbuf.at[0, pl.ds(r0, nr)],
                                    l_send.at[0, s], l_recv.at[0, s], left))
        gemm_store(xbf[...], me * m_per, m_per)

        for h in range(H):
            for s in range(SUBS):
                rr0, rnr = sub_rows(h, s, False)
                lr0, lnr = sub_rows(h, s, True)
                wait_recv(rbuf.at[h, pl.ds(rr0, rnr)], r_recv.at[h, s])
                if h + 1 < H:
                    fr0, fnr = sub_rows(h + 1, s, False)
                    sends.append(start_send(
                        rbuf.at[h, pl.ds(fr0, fnr)],
                        rbuf.at[h + 1, pl.ds(fr0, fnr)],
                        r_send.at[h + 1, s], r_recv.at[h + 1, s], right))
                wait_recv(lbuf.at[h, pl.ds(lr0, lnr)], l_recv.at[h, s])
                if h + 1 < H:
                    fr0, fnr = sub_rows(h + 1, s, True)
                    sends.append(start_send(
                        lbuf.at[h, pl.ds(fr0, fnr)],
                        lbuf.at[h + 1, pl.ds(fr0, fnr)],
                        l_send.at[h + 1, s], l_recv.at[h + 1, s], left))
            if h < H - 1:
                gemm_store(rbuf[h], ((me + N_DEV - 1 - h) % N_DEV) * m_per,
                           m_per)
                gemm_store(lbuf[h], ((me + 1 + h) % N_DEV) * m_per, m_per)
            else:
                far = ((me + N_DEV - 8) % N_DEV) * m_per
                gemm_store(rbuf[h, :half, :], far, half)
                gemm_store(lbuf[h, half:, :], far + half, half)

        for s in sends:
            s.wait_send()

        if _KMODE >= 2:
            return

        local_amax = jnp.max(out_ref[...])
        a_sends = []
        if _KMODE == 0:
            abuf[0] = jnp.full(abuf.shape[1:], local_amax, jnp.float32)
            for kk in range(1, N_DEV):
                tgt = (me + kk) % N_DEV
                a_sends.append(start_send(abuf.at[0], abuf.at[N_DEV - kk],
                                          a_send.at[kk],
                                          a_recv.at[N_DEV - kk], tgt))
            for j in range(1, N_DEV):
                wait_recv(abuf.at[j], a_recv.at[j])
            global_amax = jnp.max(abuf[...])
        else:
            global_amax = local_amax

        scale = global_amax * (1.0 / 448.0)
        q = jnp.minimum(out_ref[...] * (1.0 / scale), 448.0)
        out_ref[...] = q.astype(jnp.float8_e4m3fn).astype(jnp.float32) * scale

        for s in a_sends:
            s.wait_send()

    return pl.pallas_call(
        body,
        out_shape=jax.ShapeDtypeStruct((N_DEV * m_per, n_per), jnp.float32),
        in_specs=[pl.BlockSpec(memory_space=pltpu.VMEM),
                  pl.BlockSpec(memory_space=pltpu.VMEM)],
        out_specs=pl.BlockSpec(memory_space=pltpu.VMEM),
        scratch_shapes=[
            pltpu.VMEM((m_per, k), jnp.bfloat16),
            pltpu.VMEM((k, n_per), jnp.bfloat16),
            pltpu.VMEM((H, m_per, k), jnp.bfloat16),
            pltpu.VMEM((H, m_per, k), jnp.bfloat16),
            pltpu.VMEM((N_DEV, 8, 128), jnp.float32),
            pltpu.SemaphoreType.DMA((H, SUBS)),
            pltpu.SemaphoreType.DMA((H, SUBS)),
            pltpu.SemaphoreType.DMA((H, SUBS)),
            pltpu.SemaphoreType.DMA((H, SUBS)),
            pltpu.SemaphoreType.DMA((N_DEV,)),
            pltpu.SemaphoreType.DMA((N_DEV,)),
        ],
        compiler_params=pltpu.CompilerParams(
            collective_id=0, vmem_limit_bytes=100 * 1024 * 1024),
    )(x, w_mat)


# device time: 188269 ns/iter; 1.0146x vs baseline; 1.0146x over previous
import functools
import os

import jax
import jax.numpy as jnp
from jax import lax
from jax.experimental import pallas as pl
from jax.experimental.pallas import tpu as pltpu

N_DEV = 16
H = 8
SUBS = int(os.environ.get("KSUBS", "2"))

CYC = [0, 4, 8, 12, 15, 11, 7, 3, 2, 6, 10, 14, 13, 9, 5, 1]
K_OF = [CYC.index(i) for i in range(N_DEV)]

_KMODE = int(os.environ.get("KMODE", "0"))


def kernel(x, w_mat):
    m_per, k = x.shape
    _, n_per = w_mat.shape

    def body(x_ref, w_ref, out_ref, xbf, wbf, rbuf, lbuf, abuf,
             r_send, r_recv, l_send, l_recv, a_send, a_recv):
        me = lax.axis_index("i")

        def tlookup(table, idx):
            out = jnp.int32(table[0])
            for i in range(1, N_DEV):
                out = jnp.where(idx == i, jnp.int32(table[i]), out)
            return out

        k = tlookup(K_OF, me)
        right = tlookup([CYC[(K_OF[i] + 1) % N_DEV] for i in range(N_DEV)], me)
        left = tlookup([CYC[(K_OF[i] - 1) % N_DEV] for i in range(N_DEV)], me)

        barrier = pltpu.get_barrier_semaphore()
        for nbr in (left, right):
            pl.semaphore_signal(barrier, inc=1, device_id=(nbr,),
                                device_id_type=pl.DeviceIdType.MESH)
        pl.semaphore_wait(barrier, 2)

        xbf[...] = x_ref[...].astype(jnp.bfloat16)
        wbf[...] = w_ref[...].astype(jnp.bfloat16)

        def start_send(src, dst, ssem, rsem, dev):
            rdma = pltpu.make_async_remote_copy(
                src_ref=src, dst_ref=dst, send_sem=ssem, recv_sem=rsem,
                device_id=(dev,), device_id_type=pl.DeviceIdType.MESH)
            rdma.start()
            return rdma

        def wait_recv(dst, rsem):
            pltpu.make_async_remote_copy(
                src_ref=dst, dst_ref=dst, send_sem=rsem, recv_sem=rsem,
                device_id=(me,), device_id_type=pl.DeviceIdType.MESH,
            ).wait_recv()

        half = m_per // 2

        amax_parts = []

        def gemm_store(chunk, row0, rows):
            if _KMODE >= 2:
                return
            y = jnp.dot(chunk, wbf[...], preferred_element_type=jnp.float32)
            y = jnp.maximum(y, 0.0)
            out_ref[pl.ds(row0, rows), :] = y
            amax_parts.append(jnp.max(y))

        def sub_rows(h, s, is_left):
            if h < H - 1:
                sub = m_per // SUBS
                return ((SUBS - 1 - s) if is_left else s) * sub, sub
            q = half // SUBS
            if is_left:
                return m_per - (s + 1) * q, q
            return s * q, q

        sends = []
        for s in range(SUBS):
            r0, nr = sub_rows(0, s, False)
            sends.append(start_send(xbf.at[pl.ds(r0, nr)],
                                    rbuf.at[0, pl.ds(r0, nr)],
                                    r_send.at[0, s], r_recv.at[0, s], right))
            r0, nr = sub_rows(0, s, True)
            sends.append(start_send(xbf.at[pl.ds(r0, nr)],
                                    lbuf.at[0, pl.ds(r0, nr)],
                                    l_send.at[0, s], l_recv.at[0, s], left))
        gemm_store(xbf[...], me * m_per, m_per)

        for h in range(H):
            for s in range(SUBS):
                rr0, rnr = sub_rows(h, s, False)
                lr0, lnr = sub_rows(h, s, True)
                wait_recv(rbuf.at[h, pl.ds(rr0, rnr)], r_recv.at[h, s])
                if h + 1 < H:
                    fr0, fnr = sub_rows(h + 1, s, False)
                    sends.append(start_send(
                        rbuf.at[h, pl.ds(fr0, fnr)],
                        rbuf.at[h + 1, pl.ds(fr0, fnr)],
                        r_send.at[h + 1, s], r_recv.at[h + 1, s], right))
                wait_recv(lbuf.at[h, pl.ds(lr0, lnr)], l_recv.at[h, s])
                if h + 1 < H:
                    fr0, fnr = sub_rows(h + 1, s, True)
                    sends.append(start_send(
                        lbuf.at[h, pl.ds(fr0, fnr)],
                        lbuf.at[h + 1, pl.ds(fr0, fnr)],
                        l_send.at[h + 1, s], l_recv.at[h + 1, s], left))
            if h < H - 1:
                org_r = tlookup(CYC, (k + N_DEV - 1 - h) % N_DEV)
                org_l = tlookup(CYC, (k + 1 + h) % N_DEV)
                gemm_store(rbuf[h], org_r * m_per, m_per)
                gemm_store(lbuf[h], org_l * m_per, m_per)
            else:
                far = tlookup(CYC, (k + 8) % N_DEV) * m_per
                gemm_store(rbuf[h, :half, :], far, half)
                gemm_store(lbuf[h, half:, :], far + half, half)

        for s in sends:
            s.wait_send()

        if _KMODE >= 2:
            return

        local_amax = functools.reduce(jnp.maximum, amax_parts)
        a_sends = []
        if _KMODE == 0:
            abuf[0] = jnp.full(abuf.shape[1:], local_amax, jnp.float32)
            for kk in range(1, N_DEV):
                tgt = (me + kk) % N_DEV
                a_sends.append(start_send(abuf.at[0], abuf.at[N_DEV - kk],
                                          a_send.at[kk],
                                          a_recv.at[N_DEV - kk], tgt))
            for j in range(1, N_DEV):
                wait_recv(abuf.at[j], a_recv.at[j])
            global_amax = jnp.max(abuf[...])
        else:
            global_amax = local_amax

        scale = global_amax * (1.0 / 448.0)
        q = jnp.minimum(out_ref[...] * (1.0 / scale), 448.0)
        out_ref[...] = q.astype(jnp.float8_e4m3fn).astype(jnp.float32) * scale

        for s in a_sends:
            s.wait_send()

    return pl.pallas_call(
        body,
        out_shape=jax.ShapeDtypeStruct((N_DEV * m_per, n_per), jnp.float32),
        in_specs=[pl.BlockSpec(memory_space=pltpu.VMEM),
                  pl.BlockSpec(memory_space=pltpu.VMEM)],
        out_specs=pl.BlockSpec(memory_space=pltpu.VMEM),
        scratch_shapes=[
            pltpu.VMEM((m_per, k), jnp.bfloat16),
            pltpu.VMEM((k, n_per), jnp.bfloat16),
            pltpu.VMEM((H, m_per, k), jnp.bfloat16),
            pltpu.VMEM((H, m_per, k), jnp.bfloat16),
            pltpu.VMEM((N_DEV, 8, 128), jnp.float32),
            pltpu.SemaphoreType.DMA((H, SUBS)),
            pltpu.SemaphoreType.DMA((H, SUBS)),
            pltpu.SemaphoreType.DMA((H, SUBS)),
            pltpu.SemaphoreType.DMA((H, SUBS)),
            pltpu.SemaphoreType.DMA((N_DEV,)),
            pltpu.SemaphoreType.DMA((N_DEV,)),
        ],
        compiler_params=pltpu.CompilerParams(
            collective_id=0, vmem_limit_bytes=100 * 1024 * 1024),
    )(x, w_mat)


# device time: 188204 ns/iter; 1.0149x vs baseline; 1.0003x over previous
import functools
import os

import jax
import jax.numpy as jnp
from jax import lax
from jax.experimental import pallas as pl
from jax.experimental.pallas import tpu as pltpu

N_DEV = 16
H = 8
SUBS = int(os.environ.get("KSUBS", "2"))

CYC = [0, 4, 8, 12, 15, 11, 7, 3, 2, 6, 10, 14, 13, 9, 5, 1]
K_OF = [CYC.index(i) for i in range(N_DEV)]

_KMODE = int(os.environ.get("KMODE", "0"))


def kernel(x, w_mat):
    m_per, k = x.shape
    _, n_per = w_mat.shape

    def body(x_ref, w_ref, out_ref, xbf, wbf, rbuf, lbuf, abuf,
             r_send, r_recv, l_send, l_recv, a_send, a_recv):
        me = lax.axis_index("i")

        def tlookup(table, idx):
            out = jnp.int32(table[0])
            for i in range(1, N_DEV):
                out = jnp.where(idx == i, jnp.int32(table[i]), out)
            return out

        k = tlookup(K_OF, me)
        right = tlookup([CYC[(K_OF[i] + 1) % N_DEV] for i in range(N_DEV)], me)
        left = tlookup([CYC[(K_OF[i] - 1) % N_DEV] for i in range(N_DEV)], me)

        xbf[...] = x_ref[...].astype(jnp.bfloat16)

        barrier = pltpu.get_barrier_semaphore()
        for nbr in (left, right):
            pl.semaphore_signal(barrier, inc=1, device_id=(nbr,),
                                device_id_type=pl.DeviceIdType.MESH)
        pl.semaphore_wait(barrier, 2)

        def start_send(src, dst, ssem, rsem, dev):
            rdma = pltpu.make_async_remote_copy(
                src_ref=src, dst_ref=dst, send_sem=ssem, recv_sem=rsem,
                device_id=(dev,), device_id_type=pl.DeviceIdType.MESH)
            rdma.start()
            return rdma

        def wait_recv(dst, rsem):
            pltpu.make_async_remote_copy(
                src_ref=dst, dst_ref=dst, send_sem=rsem, recv_sem=rsem,
                device_id=(me,), device_id_type=pl.DeviceIdType.MESH,
            ).wait_recv()

        half = m_per // 2

        amax_parts = []

        def gemm_store(chunk, row0, rows):
            if _KMODE >= 2:
                return
            y = jnp.dot(chunk, wbf[...], preferred_element_type=jnp.float32)
            y = jnp.maximum(y, 0.0)
            out_ref[pl.ds(row0, rows), :] = y
            amax_parts.append(jnp.max(y))

        def sub_rows(h, s, is_left):
            if h < H - 1:
                sub = m_per // SUBS
                return ((SUBS - 1 - s) if is_left else s) * sub, sub
            q = half // SUBS
            if is_left:
                return m_per - (s + 1) * q, q
            return s * q, q

        sends = []
        for s in range(SUBS):
            r0, nr = sub_rows(0, s, False)
            sends.append(start_send(xbf.at[pl.ds(r0, nr)],
                                    rbuf.at[0, pl.ds(r0, nr)],
                                    r_send.at[0, s], r_recv.at[0, s], right))
            r0, nr = sub_rows(0, s, True)
            sends.append(start_send(xbf.at[pl.ds(r0, nr)],
                                    lbuf.at[0, pl.ds(r0, nr)],
                                    l_send.at[0, s], l_recv.at[0, s], left))
        wbf[...] = w_ref[...].astype(jnp.bfloat16)
        gemm_store(xbf[...], me * m_per, m_per)

        for h in range(H):
            for s in range(SUBS):
                rr0, rnr = sub_rows(h, s, False)
                lr0, lnr = sub_rows(h, s, True)
                wait_recv(rbuf.at[h, pl.ds(rr0, rnr)], r_recv.at[h, s])
                if h + 1 < H:
                    fr0, fnr = sub_rows(h + 1, s, False)
                    sends.append(start_send(
                        rbuf.at[h, pl.ds(fr0, fnr)],
                        rbuf.at[h + 1, pl.ds(fr0, fnr)],
                        r_send.at[h + 1, s], r_recv.at[h + 1, s], right))
                wait_recv(lbuf.at[h, pl.ds(lr0, lnr)], l_recv.at[h, s])
                if h + 1 < H:
                    fr0, fnr = sub_rows(h + 1, s, True)
                    sends.append(start_send(
                        lbuf.at[h, pl.ds(fr0, fnr)],
                        lbuf.at[h + 1, pl.ds(fr0, fnr)],
                        l_send.at[h + 1, s], l_recv.at[h + 1, s], left))
            if h < H - 1:
                org_r = tlookup(CYC, (k + N_DEV - 1 - h) % N_DEV)
                org_l = tlookup(CYC, (k + 1 + h) % N_DEV)
                gemm_store(rbuf[h], org_r * m_per, m_per)
                gemm_store(lbuf[h], org_l * m_per, m_per)
            else:
                far = tlookup(CYC, (k + 8) % N_DEV) * m_per
                gemm_store(rbuf[h, :half, :], far, half)
                gemm_store(lbuf[h, half:, :], far + half, half)

        for s in sends:
            s.wait_send()

        if _KMODE >= 2:
            return

        local_amax = functools.reduce(jnp.maximum, amax_parts)
        a_sends = []
        if _KMODE == 0:
            abuf[0] = jnp.full(abuf.shape[1:], local_amax, jnp.float32)
            for kk in range(1, N_DEV):
                tgt = (me + kk) % N_DEV
                a_sends.append(start_send(abuf.at[0], abuf.at[N_DEV - kk],
                                          a_send.at[kk],
                                          a_recv.at[N_DEV - kk], tgt))
            for j in range(1, N_DEV):
                wait_recv(abuf.at[j], a_recv.at[j])
            global_amax = jnp.max(abuf[...])
        else:
            global_amax = local_amax

        scale = global_amax * (1.0 / 448.0)
        q = jnp.minimum(out_ref[...] * (1.0 / scale), 448.0)
        out_ref[...] = q.astype(jnp.float8_e4m3fn).astype(jnp.float32) * scale

        for s in a_sends:
            s.wait_send()

    return pl.pallas_call(
        body,
        out_shape=jax.ShapeDtypeStruct((N_DEV * m_per, n_per), jnp.float32),
        in_specs=[pl.BlockSpec(memory_space=pltpu.VMEM),
                  pl.BlockSpec(memory_space=pltpu.VMEM)],
        out_specs=pl.BlockSpec(memory_space=pltpu.VMEM),
        scratch_shapes=[
            pltpu.VMEM((m_per, k), jnp.bfloat16),
            pltpu.VMEM((k, n_per), jnp.bfloat16),
            pltpu.VMEM((H, m_per, k), jnp.bfloat16),
            pltpu.VMEM((H, m_per, k), jnp.bfloat16),
            pltpu.VMEM((N_DEV, 8, 128), jnp.float32),
            pltpu.SemaphoreType.DMA((H, SUBS)),
            pltpu.SemaphoreType.DMA((H, SUBS)),
            pltpu.SemaphoreType.DMA((H, SUBS)),
            pltpu.SemaphoreType.DMA((H, SUBS)),
            pltpu.SemaphoreType.DMA((N_DEV,)),
            pltpu.SemaphoreType.DMA((N_DEV,)),
        ],
        compiler_params=pltpu.CompilerParams(
            collective_id=0, vmem_limit_bytes=100 * 1024 * 1024),
    )(x, w_mat)


# device time: 188161 ns/iter; 1.0152x vs baseline; 1.0002x over previous
import functools

import jax
import jax.numpy as jnp
from jax import lax
from jax.experimental import pallas as pl
from jax.experimental.pallas import tpu as pltpu

N_DEV = 16
H = 8
SUBS = 2

CYC = [0, 4, 8, 12, 15, 11, 7, 3, 2, 6, 10, 14, 13, 9, 5, 1]
K_OF = [CYC.index(i) for i in range(N_DEV)]


def kernel(x, w_mat):
    m_per, k = x.shape
    _, n_per = w_mat.shape

    def body(x_ref, w_ref, out_ref, xbf, wbf, rbuf, lbuf, abuf,
             r_send, r_recv, l_send, l_recv, a_send, a_recv):
        me = lax.axis_index("i")

        def tlookup(table, idx):
            out = jnp.int32(table[0])
            for i in range(1, N_DEV):
                out = jnp.where(idx == i, jnp.int32(table[i]), out)
            return out

        k = tlookup(K_OF, me)
        right = tlookup([CYC[(K_OF[i] + 1) % N_DEV] for i in range(N_DEV)], me)
        left = tlookup([CYC[(K_OF[i] - 1) % N_DEV] for i in range(N_DEV)], me)

        xbf[...] = x_ref[...].astype(jnp.bfloat16)

        barrier = pltpu.get_barrier_semaphore()
        for nbr in (left, right):
            pl.semaphore_signal(barrier, inc=1, device_id=(nbr,),
                                device_id_type=pl.DeviceIdType.MESH)
        pl.semaphore_wait(barrier, 2)

        def start_send(src, dst, ssem, rsem, dev):
            rdma = pltpu.make_async_remote_copy(
                src_ref=src, dst_ref=dst, send_sem=ssem, recv_sem=rsem,
                device_id=(dev,), device_id_type=pl.DeviceIdType.MESH)
            rdma.start()
            return rdma

        def wait_recv(dst, rsem):
            pltpu.make_async_remote_copy(
                src_ref=dst, dst_ref=dst, send_sem=rsem, recv_sem=rsem,
                device_id=(me,), device_id_type=pl.DeviceIdType.MESH,
            ).wait_recv()

        half = m_per // 2

        amax_parts = []

        def gemm_store(chunk, row0, rows):
            y = jnp.dot(chunk, wbf[...], preferred_element_type=jnp.float32)
            y = jnp.maximum(y, 0.0)
            out_ref[pl.ds(row0, rows), :] = y
            amax_parts.append(jnp.max(y))

        def sub_rows(h, s, is_left):
            if h < H - 1:
                sub = m_per // SUBS
                return ((SUBS - 1 - s) if is_left else s) * sub, sub
            q = half // SUBS
            if is_left:
                return m_per - (s + 1) * q, q
            return s * q, q

        sends = []
        for s in range(SUBS):
            r0, nr = sub_rows(0, s, False)
            sends.append(start_send(xbf.at[pl.ds(r0, nr)],
                                    rbuf.at[0, pl.ds(r0, nr)],
                                    r_send.at[0, s], r_recv.at[0, s], right))
            r0, nr = sub_rows(0, s, True)
            sends.append(start_send(xbf.at[pl.ds(r0, nr)],
                                    lbuf.at[0, pl.ds(r0, nr)],
                                    l_send.at[0, s], l_recv.at[0, s], left))
        wbf[...] = w_ref[...].astype(jnp.bfloat16)
        gemm_store(xbf[...], me * m_per, m_per)

        for h in range(H):
            for s in range(SUBS):
                rr0, rnr = sub_rows(h, s, False)
                lr0, lnr = sub_rows(h, s, True)
                wait_recv(rbuf.at[h, pl.ds(rr0, rnr)], r_recv.at[h, s])
                if h + 1 < H:
                    fr0, fnr = sub_rows(h + 1, s, False)
                    sends.append(start_send(
                        rbuf.at[h, pl.ds(fr0, fnr)],
                        rbuf.at[h + 1, pl.ds(fr0, fnr)],
                        r_send.at[h + 1, s], r_recv.at[h + 1, s], right))
                wait_recv(lbuf.at[h, pl.ds(lr0, lnr)], l_recv.at[h, s])
                if h + 1 < H:
                    fr0, fnr = sub_rows(h + 1, s, True)
                    sends.append(start_send(
                        lbuf.at[h, pl.ds(fr0, fnr)],
                        lbuf.at[h + 1, pl.ds(fr0, fnr)],
                        l_send.at[h + 1, s], l_recv.at[h + 1, s], left))
            if h < H - 1:
                org_r = tlookup(CYC, (k + N_DEV - 1 - h) % N_DEV)
                org_l = tlookup(CYC, (k + 1 + h) % N_DEV)
                gemm_store(rbuf[h], org_r * m_per, m_per)
                gemm_store(lbuf[h], org_l * m_per, m_per)
            else:
                far = tlookup(CYC, (k + 8) % N_DEV) * m_per
                gemm_store(rbuf[h, :half, :], far, half)
                gemm_store(lbuf[h, half:, :], far + half, half)

        for s in sends:
            s.wait_send()

        local_amax = functools.reduce(jnp.maximum, amax_parts)
        a_sends = []
        abuf[0] = jnp.full(abuf.shape[1:], local_amax, jnp.float32)
        for kk in range(1, N_DEV):
            tgt = (me + kk) % N_DEV
            a_sends.append(start_send(abuf.at[0], abuf.at[N_DEV - kk],
                                      a_send.at[kk],
                                      a_recv.at[N_DEV - kk], tgt))
        for j in range(1, N_DEV):
            wait_recv(abuf.at[j], a_recv.at[j])
        global_amax = jnp.max(abuf[...])

        scale = global_amax * (1.0 / 448.0)
        q = jnp.minimum(out_ref[...] * (1.0 / scale), 448.0)
        out_ref[...] = q.astype(jnp.float8_e4m3fn).astype(jnp.float32) * scale

        for s in a_sends:
            s.wait_send()

    return pl.pallas_call(
        body,
        out_shape=jax.ShapeDtypeStruct((N_DEV * m_per, n_per), jnp.float32),
        in_specs=[pl.BlockSpec(memory_space=pltpu.VMEM),
                  pl.BlockSpec(memory_space=pltpu.VMEM)],
        out_specs=pl.BlockSpec(memory_space=pltpu.VMEM),
        scratch_shapes=[
            pltpu.VMEM((m_per, k), jnp.bfloat16),
            pltpu.VMEM((k, n_per), jnp.bfloat16),
            pltpu.VMEM((H, m_per, k), jnp.bfloat16),
            pltpu.VMEM((H, m_per, k), jnp.bfloat16),
            pltpu.VMEM((N_DEV, 8, 128), jnp.float32),
            pltpu.SemaphoreType.DMA((H, SUBS)),
            pltpu.SemaphoreType.DMA((H, SUBS)),
            pltpu.SemaphoreType.DMA((H, SUBS)),
            pltpu.SemaphoreType.DMA((H, SUBS)),
            pltpu.SemaphoreType.DMA((N_DEV,)),
            pltpu.SemaphoreType.DMA((N_DEV,)),
        ],
        compiler_params=pltpu.CompilerParams(
            collective_id=0, vmem_limit_bytes=100 * 1024 * 1024),
    )(x, w_mat)
